# baseline (device time: 28556 ns/iter reference)
import jax
import jax.numpy as jnp
from jax import lax
from jax.experimental import pallas as pl
from jax.experimental.pallas import tpu as pltpu

N_DEV = 4
N_TOK = 1024
D_IN = 512
D_OUT = 1024
E_LOCAL = 4
CAP = 51
CHUNK = N_TOK // N_DEV


def kernel(x, router_W, route_idx, expert_W):
    del router_W

    my = lax.axis_index("i")
    e_ids = my * E_LOCAL + jnp.arange(E_LOCAL, dtype=jnp.int32)
    oh = (route_idx == e_ids[None, :]).astype(jnp.float32)
    pos = jnp.cumsum(oh, axis=0) - oh
    mask = oh * (pos < float(CAP)).astype(jnp.float32)

    def body(x_ref, mask_ref, w_ref, out_ref, send_ref, recv_ref, send_sems, recv_sems):
        me = lax.axis_index("i")

        barrier_sem = pltpu.get_barrier_semaphore()
        for k in range(1, N_DEV):
            pl.semaphore_signal(
                barrier_sem, inc=1,
                device_id=((me + k) % N_DEV,),
                device_id_type=pl.DeviceIdType.MESH,
            )
        pl.semaphore_wait(barrier_sem, N_DEV - 1)

        wks = [w_ref[k].astype(jnp.bfloat16) for k in range(E_LOCAL)]

        def compute_chunk(j):
            xc = x_ref[pl.ds(j * CHUNK, CHUNK), :]
            mc = mask_ref[pl.ds(j * CHUNK, CHUNK), :]
            acc = jnp.zeros((CHUNK, D_OUT), dtype=jnp.float32)
            for k in range(E_LOCAL):
                xm = (xc * mc[:, k:k + 1]).astype(jnp.bfloat16)
                acc = acc + jnp.dot(xm, wks[k], preferred_element_type=jnp.float32)
            return acc

        rdmas = []
        for k in range(1, N_DEV):
            tgt = (me + k) % N_DEV
            send_ref[k - 1, :, :] = compute_chunk(tgt).astype(jnp.bfloat16)
            rdma = pltpu.make_async_remote_copy(
                src_ref=send_ref.at[k - 1],
                dst_ref=recv_ref.at[k - 1],
                send_sem=send_sems.at[k - 1],
                recv_sem=recv_sems.at[k - 1],
                device_id=(tgt,),
                device_id_type=pl.DeviceIdType.MESH,
            )
            rdma.start()
            rdmas.append(rdma)

        own = compute_chunk(me)

        for s in range(N_DEV - 1):
            rdmas[s].wait_recv()
            own = own + recv_ref[s, :, :].astype(jnp.float32)
        out_ref[...] = own

        for s in range(N_DEV - 1):
            rdmas[s].wait_send()

    return pl.pallas_call(
        body,
        out_shape=jax.ShapeDtypeStruct((CHUNK, D_OUT), jnp.float32),
        in_specs=[
            pl.BlockSpec(memory_space=pltpu.VMEM),
            pl.BlockSpec(memory_space=pltpu.VMEM),
            pl.BlockSpec(memory_space=pltpu.VMEM),
        ],
        out_specs=pl.BlockSpec(memory_space=pltpu.VMEM),
        scratch_shapes=[
            pltpu.VMEM((N_DEV - 1, CHUNK, D_OUT), jnp.bfloat16),
            pltpu.VMEM((N_DEV - 1, CHUNK, D_OUT), jnp.bfloat16),
            pltpu.SemaphoreType.DMA((N_DEV - 1,)),
            pltpu.SemaphoreType.DMA((N_DEV - 1,)),
        ],
        compiler_params=pltpu.CompilerParams(collective_id=0),
    )(x, mask, expert_W)


# device time: 28533 ns/iter; 1.0008x vs baseline; 1.0008x over previous
import jax
import jax.numpy as jnp
from jax import lax
from jax.experimental import pallas as pl
from jax.experimental.pallas import tpu as pltpu

N_DEV = 4
N_TOK = 1024
D_IN = 512
D_OUT = 1024
E_LOCAL = 4
CAP = 51
CHUNK = N_TOK // N_DEV


def kernel(x, router_W, route_idx, expert_W):
    del router_W

    my = lax.axis_index("i")
    e_ids = my * E_LOCAL + jnp.arange(E_LOCAL, dtype=jnp.int32)
    oh = (route_idx == e_ids[None, :]).astype(jnp.float32)
    pos = jnp.cumsum(oh, axis=0) - oh
    mask = oh * (pos < float(CAP)).astype(jnp.float32)

    def body(x_ref, mask_ref, w_ref, out_ref, send_ref, recv_ref, send_sems, recv_sems):
        me = lax.axis_index("i")

        barrier_sem = pltpu.get_barrier_semaphore()
        for k in range(1, N_DEV):
            pl.semaphore_signal(
                barrier_sem, inc=1,
                device_id=((me + k) % N_DEV,),
                device_id_type=pl.DeviceIdType.MESH,
            )
        pl.semaphore_wait(barrier_sem, N_DEV - 1)

        w_cat = w_ref[...].reshape(E_LOCAL * D_IN, D_OUT).astype(jnp.bfloat16)

        def compute_chunk(j):
            xc = x_ref[pl.ds(j * CHUNK, CHUNK), :]
            mc = mask_ref[pl.ds(j * CHUNK, CHUNK), :]
            xm = jnp.concatenate(
                [(xc * mc[:, k:k + 1]).astype(jnp.bfloat16) for k in range(E_LOCAL)],
                axis=1,
            )
            return jnp.dot(xm, w_cat, preferred_element_type=jnp.float32)

        rdmas = []
        for k in range(1, N_DEV):
            tgt = (me + k) % N_DEV
            send_ref[k - 1, :, :] = compute_chunk(tgt).astype(jnp.bfloat16)
            rdma = pltpu.make_async_remote_copy(
                src_ref=send_ref.at[k - 1],
                dst_ref=recv_ref.at[k - 1],
                send_sem=send_sems.at[k - 1],
                recv_sem=recv_sems.at[k - 1],
                device_id=(tgt,),
                device_id_type=pl.DeviceIdType.MESH,
            )
            rdma.start()
            rdmas.append(rdma)

        own = compute_chunk(me)

        for s in range(N_DEV - 1):
            rdmas[s].wait_recv()
            own = own + recv_ref[s, :, :].astype(jnp.float32)
        out_ref[...] = own

        for s in range(N_DEV - 1):
            rdmas[s].wait_send()

    return pl.pallas_call(
        body,
        out_shape=jax.ShapeDtypeStruct((CHUNK, D_OUT), jnp.float32),
        in_specs=[
            pl.BlockSpec(memory_space=pltpu.VMEM),
            pl.BlockSpec(memory_space=pltpu.VMEM),
            pl.BlockSpec(memory_space=pltpu.VMEM),
        ],
        out_specs=pl.BlockSpec(memory_space=pltpu.VMEM),
        scratch_shapes=[
            pltpu.VMEM((N_DEV - 1, CHUNK, D_OUT), jnp.bfloat16),
            pltpu.VMEM((N_DEV - 1, CHUNK, D_OUT), jnp.bfloat16),
            pltpu.SemaphoreType.DMA((N_DEV - 1,)),
            pltpu.SemaphoreType.DMA((N_DEV - 1,)),
        ],
        compiler_params=pltpu.CompilerParams(collective_id=0),
    )(x, mask, expert_W)


# device time: 27861 ns/iter; 1.0249x vs baseline; 1.0241x over previous
import jax
import jax.numpy as jnp
from jax import lax
from jax.experimental import pallas as pl
from jax.experimental.pallas import tpu as pltpu

N_DEV = 4
N_TOK = 1024
D_IN = 512
D_OUT = 1024
E_LOCAL = 4
CAP = 51
CHUNK = N_TOK // N_DEV


def kernel(x, router_W, route_idx, expert_W):
    del router_W

    def body(x_ref, idx_ref, w_ref, out_ref, mask_ref, send_ref, recv_ref,
             send_sems, recv_sems):
        me = lax.axis_index("i")

        barrier_sem = pltpu.get_barrier_semaphore()
        for k in range(1, N_DEV):
            pl.semaphore_signal(
                barrier_sem, inc=1,
                device_id=((me + k) % N_DEV,),
                device_id_type=pl.DeviceIdType.MESH,
            )
        pl.semaphore_wait(barrier_sem, N_DEV - 1)

        e_row = me * E_LOCAL + lax.broadcasted_iota(jnp.int32, (1, E_LOCAL), 1)
        oh = (idx_ref[...] == e_row).astype(jnp.bfloat16)
        row_i = lax.broadcasted_iota(jnp.int32, (N_TOK, N_TOK), 0)
        col_i = lax.broadcasted_iota(jnp.int32, (N_TOK, N_TOK), 1)
        tri_l = (col_i < row_i).astype(jnp.bfloat16)
        pos = jnp.dot(tri_l, oh, preferred_element_type=jnp.float32)
        mask_ref[...] = oh.astype(jnp.float32) * (pos < float(CAP)).astype(
            jnp.float32
        )

        w_cat = w_ref[...].reshape(E_LOCAL * D_IN, D_OUT).astype(jnp.bfloat16)

        def compute_chunk(j):
            xc = x_ref[pl.ds(j * CHUNK, CHUNK), :]
            mc = mask_ref[pl.ds(j * CHUNK, CHUNK), :]
            xm = jnp.concatenate(
                [(xc * mc[:, k:k + 1]).astype(jnp.bfloat16) for k in range(E_LOCAL)],
                axis=1,
            )
            return jnp.dot(xm, w_cat, preferred_element_type=jnp.float32)

        rdmas = []
        for k in range(1, N_DEV):
            tgt = (me + k) % N_DEV
            send_ref[k - 1, :, :] = compute_chunk(tgt).astype(jnp.bfloat16)
            rdma = pltpu.make_async_remote_copy(
                src_ref=send_ref.at[k - 1],
                dst_ref=recv_ref.at[k - 1],
                send_sem=send_sems.at[k - 1],
                recv_sem=recv_sems.at[k - 1],
                device_id=(tgt,),
                device_id_type=pl.DeviceIdType.MESH,
            )
            rdma.start()
            rdmas.append(rdma)

        own = compute_chunk(me)

        for s in range(N_DEV - 1):
            rdmas[s].wait_recv()
            own = own + recv_ref[s, :, :].astype(jnp.float32)
        out_ref[...] = own

        for s in range(N_DEV - 1):
            rdmas[s].wait_send()

    return pl.pallas_call(
        body,
        out_shape=jax.ShapeDtypeStruct((CHUNK, D_OUT), jnp.float32),
        in_specs=[
            pl.BlockSpec(memory_space=pltpu.VMEM),
            pl.BlockSpec(memory_space=pltpu.VMEM),
            pl.BlockSpec(memory_space=pltpu.VMEM),
        ],
        out_specs=pl.BlockSpec(memory_space=pltpu.VMEM),
        scratch_shapes=[
            pltpu.VMEM((N_TOK, E_LOCAL), jnp.float32),
            pltpu.VMEM((N_DEV - 1, CHUNK, D_OUT), jnp.bfloat16),
            pltpu.VMEM((N_DEV - 1, CHUNK, D_OUT), jnp.bfloat16),
            pltpu.SemaphoreType.DMA((N_DEV - 1,)),
            pltpu.SemaphoreType.DMA((N_DEV - 1,)),
        ],
        compiler_params=pltpu.CompilerParams(collective_id=0),
    )(x, route_idx, expert_W)
